# baseline (speedup 1.0000x reference)
import functools

import jax
import jax.numpy as jnp
from jax import lax
from jax.experimental import pallas as pl
from jax.experimental.pallas import tpu as pltpu
from jax.experimental.pallas import tpu_sc as plsc

N = 90000
ROIS = 90
BATCH = N // ROIS
E = 2880000
H0 = 3
HID = 16
NLAYERS = 3
HL = 64
NCLS = 2

SC_CORES = 2
SC_SUBCORES = 16
NW = SC_CORES * SC_SUBCORES
EPW = E // NW
CHUNK = 3600
NCH = EPW // CHUNK
NPS = N // SC_SUBCORES
N1 = 90112
NPS1 = N1 // SC_SUBCORES
HALF = HID // 2

_mesh = plsc.VectorSubcoreMesh(
    core_axis_name="c", subcore_axis_name="s",
    num_cores=SC_CORES, num_subcores=SC_SUBCORES)


def _worker_id():
    cid = lax.axis_index("c")
    sid = lax.axis_index("s")
    return cid, sid, sid * SC_CORES + cid


@functools.partial(
    pl.kernel, mesh=_mesh,
    compiler_params=pltpu.CompilerParams(use_tc_tiling_on_sc=False),
    out_type=(jax.ShapeDtypeStruct((E,), jnp.float32),
              jax.ShapeDtypeStruct((SC_CORES, N1), jnp.float32)),
    scratch_types=[
        pltpu.VMEM_SHARED((N1,), jnp.float32),
        pltpu.VMEM((CHUNK,), jnp.int32),
        pltpu.VMEM((CHUNK,), jnp.int32),
        pltpu.VMEM((CHUNK,), jnp.float32),
        pltpu.VMEM((CHUNK,), jnp.float32),
        pltpu.VMEM((CHUNK,), jnp.float32),
        pltpu.VMEM((CHUNK,), jnp.float32),
        pltpu.VMEM((CHUNK,), jnp.int32),
        pltpu.VMEM((CHUNK,), jnp.int32),
        pltpu.VMEM((CHUNK,), jnp.float32),
        pltpu.VMEM((CHUNK,), jnp.float32),
        pltpu.VMEM((CHUNK,), jnp.float32),
        pltpu.VMEM((CHUNK,), jnp.float32),
        pltpu.SemaphoreType.DMA,
        pltpu.SemaphoreType.DMA,
        pltpu.SemaphoreType.DMA,
        pltpu.SemaphoreType.DMA,
        pltpu.SemaphoreType.DMA,
        pltpu.SemaphoreType.DMA,
        pltpu.SemaphoreType.DMA,
        pltpu.SemaphoreType.DMA,
    ],
)
def _sc_edge_prob(a_hbm, b_hbm, ei_hbm, ew_hbm, zeros_hbm,
                  w_hbm, deg_hbm,
                  deg_sp,
                  sv0, dv0, ev0, ag0, bg0, wv0,
                  sv1, dv1, ev1, ag1, bg1, wv1,
                  semI0, semI1, semG0, semG1, semW0, semW1, semD0, semD1):
    cid, sid, wid = _worker_id()
    base = wid * EPW
    bufs = ((sv0, dv0, ev0, ag0, bg0, wv0, semI0, semG0, semW0, semD0),
            (sv1, dv1, ev1, ag1, bg1, wv1, semI1, semG1, semW1, semD1))

    def issue_idx(cc, b):
        sv, dv, ev, _, _, _, semI, _, _, _ = bufs[b]
        off = base + cc * CHUNK
        pltpu.async_copy(ei_hbm.at[0, pl.ds(off, CHUNK)], sv, semI)
        pltpu.async_copy(ei_hbm.at[1, pl.ds(off, CHUNK)], dv, semI)
        pltpu.async_copy(ew_hbm.at[pl.ds(off, CHUNK)], ev, semI)

    def wait_idx(cc, b):
        sv, dv, ev, _, _, _, semI, _, _, _ = bufs[b]
        off = base + cc * CHUNK
        pltpu.make_async_copy(ei_hbm.at[0, pl.ds(off, CHUNK)], sv, semI).wait()
        pltpu.make_async_copy(ei_hbm.at[1, pl.ds(off, CHUNK)], dv, semI).wait()
        pltpu.make_async_copy(ew_hbm.at[pl.ds(off, CHUNK)], ev, semI).wait()

    def issue_gathers(b):
        sv, dv, _, ag, bg, _, _, semG, _, _ = bufs[b]
        pltpu.async_copy(a_hbm.at[sv], ag, semG)
        pltpu.async_copy(b_hbm.at[dv], bg, semG)

    def wait_gathers(b):
        sv, dv, _, ag, bg, _, _, semG, _, _ = bufs[b]
        pltpu.make_async_copy(a_hbm.at[sv], ag, semG).wait()
        pltpu.make_async_copy(b_hbm.at[dv], bg, semG).wait()

    def issue_w(cc, b):
        _, _, _, _, _, wv, _, _, semW, _ = bufs[b]
        off = base + cc * CHUNK
        pltpu.async_copy(wv, w_hbm.at[pl.ds(off, CHUNK)], semW)

    def wait_w(cc, b):
        _, _, _, _, _, wv, _, _, semW, _ = bufs[b]
        off = base + cc * CHUNK
        pltpu.make_async_copy(wv, w_hbm.at[pl.ds(off, CHUNK)], semW).wait()

    def issue_d(b):
        _, dv, _, _, _, wv, _, _, _, semD = bufs[b]
        pltpu.async_copy(wv, deg_sp.at[dv], semD, add=True)

    def wait_d(b):
        _, dv, _, _, _, wv, _, _, _, semD = bufs[b]
        pltpu.make_async_copy(wv, deg_sp.at[dv], semD).wait()

    def compute(b):
        _, _, ev, ag, bg, wv, _, _, _, _ = bufs[b]

        def vec_body(k, c2):
            s = pl.ds(k * 16, 16)
            t = ag[s] + bg[s]
            p = 1.0 / (1.0 + jnp.exp(-t))
            wv[s] = ev[s] * p
            return c2

        lax.fori_loop(0, CHUNK // 16, vec_body, 0)

    pltpu.sync_copy(zeros_hbm.at[pl.ds(sid * NPS1, NPS1)],
                    deg_sp.at[pl.ds(sid * NPS1, NPS1)])
    plsc.subcore_barrier()

    issue_idx(0, 0)
    wait_idx(0, 0)
    issue_gathers(0)

    def pair_body(j, carry):
        @pl.when(j >= 1)
        def _():
            wait_d(1)
            wait_w(2 * j - 1, 1)
        issue_idx(2 * j + 1, 1)
        wait_gathers(0)
        compute(0)
        wait_idx(2 * j + 1, 1)
        issue_gathers(1)
        issue_w(2 * j, 0)
        issue_d(0)
        wait_d(0)
        wait_w(2 * j, 0)
        issue_idx(2 * j + 2, 0)
        wait_gathers(1)
        compute(1)
        wait_idx(2 * j + 2, 0)
        issue_gathers(0)
        issue_w(2 * j + 1, 1)
        issue_d(1)
        return carry

    lax.fori_loop(0, (NCH - 1) // 2, pair_body, 0)
    wait_d(1)
    wait_w(NCH - 2, 1)
    wait_gathers(0)
    compute(0)
    issue_w(NCH - 1, 0)
    issue_d(0)
    wait_d(0)
    wait_w(NCH - 1, 0)
    plsc.subcore_barrier()
    pltpu.sync_copy(deg_sp.at[pl.ds(sid * NPS1, NPS1)],
                    deg_hbm.at[cid, pl.ds(sid * NPS1, NPS1)])


CHUNK_G = 720
NCH_G = EPW // CHUNK_G


@functools.partial(
    pl.kernel, mesh=_mesh,
    compiler_params=pltpu.CompilerParams(use_tc_tiling_on_sc=False),
    out_type=jax.ShapeDtypeStruct((SC_CORES, N1, HID), jnp.float32),
    scratch_types=[
        pltpu.VMEM_SHARED((N1, HID), jnp.float32),
        pltpu.VMEM((CHUNK_G,), jnp.int32),
        pltpu.VMEM((CHUNK_G,), jnp.int32),
        pltpu.VMEM((CHUNK_G,), jnp.float32),
        pltpu.VMEM((CHUNK_G, HID), jnp.float32),
        pltpu.VMEM((CHUNK_G,), jnp.int32),
        pltpu.VMEM((CHUNK_G,), jnp.int32),
        pltpu.VMEM((CHUNK_G,), jnp.float32),
        pltpu.VMEM((CHUNK_G, HID), jnp.float32),
        pltpu.SemaphoreType.DMA,
        pltpu.SemaphoreType.DMA,
        pltpu.SemaphoreType.DMA,
        pltpu.SemaphoreType.DMA,
        pltpu.SemaphoreType.DMA,
        pltpu.SemaphoreType.DMA,
    ],
)
def _sc_agg(h_hbm, ei_hbm, norm_hbm, zseg_hbm,
            part_hbm,
            acc_sp, sv0, dv0, nv0, rows0, sv1, dv1, nv1, rows1,
            semI0, semI1, semG0, semG1, semS0, semS1):
    cid, sid, wid = _worker_id()
    base = wid * EPW
    bufs = ((sv0, dv0, nv0, rows0, semI0, semG0, semS0),
            (sv1, dv1, nv1, rows1, semI1, semG1, semS1))

    def issue_idx(cc, b):
        sv, dv, nv, _, semI, _, _ = bufs[b]
        off = base + cc * CHUNK_G
        pltpu.async_copy(ei_hbm.at[0, pl.ds(off, CHUNK_G)], sv, semI)
        pltpu.async_copy(ei_hbm.at[1, pl.ds(off, CHUNK_G)], dv, semI)
        pltpu.async_copy(norm_hbm.at[pl.ds(off, CHUNK_G)], nv, semI)

    def wait_idx(cc, b):
        sv, dv, nv, _, semI, _, _ = bufs[b]
        off = base + cc * CHUNK_G
        pltpu.make_async_copy(ei_hbm.at[0, pl.ds(off, CHUNK_G)], sv, semI).wait()
        pltpu.make_async_copy(ei_hbm.at[1, pl.ds(off, CHUNK_G)], dv, semI).wait()
        pltpu.make_async_copy(norm_hbm.at[pl.ds(off, CHUNK_G)], nv, semI).wait()

    def issue_gather(b):
        sv, _, _, rows, _, semG, _ = bufs[b]
        pltpu.async_copy(h_hbm.at[sv], rows, semG)

    def wait_gather(b):
        sv, _, _, rows, _, semG, _ = bufs[b]
        pltpu.make_async_copy(h_hbm.at[sv], rows, semG).wait()

    def issue_scat(b):
        _, dv, _, rows, _, _, semS = bufs[b]
        pltpu.async_copy(rows, acc_sp.at[dv], semS, add=True)

    def wait_scat(b):
        _, dv, _, rows, _, _, semS = bufs[b]
        pltpu.make_async_copy(rows, acc_sp.at[dv], semS).wait()

    def scale(b):
        _, _, nv, rows, _, _, _ = bufs[b]

        def scale_body(k, c2):
            nvec = nv[pl.ds(k * 16, 16)]
            for i in range(16):
                e = k * 16 + i
                rows[e] = rows[e] * jnp.full((16,), nvec[i], jnp.float32)
            return c2

        lax.fori_loop(0, CHUNK_G // 16, scale_body, 0)

    pltpu.sync_copy(zseg_hbm.at[pl.ds(sid * NPS1, NPS1)],
                    acc_sp.at[pl.ds(sid * NPS1, NPS1)])
    plsc.subcore_barrier()

    issue_idx(0, 0)
    wait_idx(0, 0)
    issue_gather(0)

    def pair_body(j, carry):
        @pl.when(j >= 1)
        def _():
            wait_scat(1)
        issue_idx(2 * j + 1, 1)
        wait_gather(0)
        scale(0)
        wait_idx(2 * j + 1, 1)
        issue_gather(1)
        issue_scat(0)
        wait_scat(0)
        issue_idx(2 * j + 2, 0)
        wait_gather(1)
        scale(1)
        wait_idx(2 * j + 2, 0)
        issue_gather(0)
        issue_scat(1)
        return carry

    lax.fori_loop(0, (NCH_G - 1) // 2, pair_body, 0)
    wait_scat(1)
    wait_gather(0)
    scale(0)
    issue_scat(0)
    wait_scat(0)
    plsc.subcore_barrier()
    pltpu.sync_copy(acc_sp.at[pl.ds(sid * NPS1, NPS1)],
                    part_hbm.at[cid, pl.ds(sid * NPS1, NPS1)])


def kernel(x, edge_index, edge_weight, temperature, prob, prob_bias,
           W1, b1, W2, b2, W3, b3, lin1_W, lin1_b, lin2_W, lin2_b):
    xf = (x.reshape(BATCH, ROIS, H0) * prob[None, :, :]).reshape(N, H0)
    pb = prob_bias[:, 0]
    a = xf @ pb[:H0]
    bvec = xf @ pb[H0:]
    zeros1 = jnp.zeros((N1,), jnp.float32)
    zseg = jnp.zeros((N1, HID), jnp.float32)

    w, deg2 = _sc_edge_prob(a, bvec, edge_index, edge_weight, zeros1)
    deg = deg2[0, :N] + deg2[1, :N] + 1.0
    dis = jnp.where(deg > 0, lax.rsqrt(jnp.maximum(deg, 1e-12)), 0.0)
    dis2 = dis * dis

    xs = []
    xk = xf
    for W, b in ((W1, b1), (W2, b2), (W3, b3)):
        h = xk @ W
        hd = dis[:, None] * h
        part = _sc_agg(hd, edge_index, w, zseg)
        acc = (part[0] + part[1])[:N]
        xk = jax.nn.relu(dis[:, None] * (acc + hd) + b)
        xs.append(xk)

    xcat = jnp.concatenate(xs, axis=-1)
    feat = xcat.reshape(BATCH, ROIS * NLAYERS * HID)
    hdense = jax.nn.relu(feat @ lin1_W + lin1_b)
    return hdense @ lin2_W + lin2_b


# speedup vs baseline: 1.1480x; 1.1480x over previous
import functools

import jax
import jax.numpy as jnp
from jax import lax
from jax.experimental import pallas as pl
from jax.experimental.pallas import tpu as pltpu
from jax.experimental.pallas import tpu_sc as plsc

N = 90000
ROIS = 90
BATCH = N // ROIS
E = 2880000
H0 = 3
HID = 16
NLAYERS = 3
HL = 64
NCLS = 2

SC_CORES = 2
SC_SUBCORES = 16
NW = SC_CORES * SC_SUBCORES
EPW = E // NW
CHUNK = 6000
NCH = EPW // CHUNK
NPS = N // SC_SUBCORES
N1 = 90112
NPS1 = N1 // SC_SUBCORES
HALF = HID // 2

_mesh = plsc.VectorSubcoreMesh(
    core_axis_name="c", subcore_axis_name="s",
    num_cores=SC_CORES, num_subcores=SC_SUBCORES)


def _worker_id():
    cid = lax.axis_index("c")
    sid = lax.axis_index("s")
    return cid, sid, sid * SC_CORES + cid


@functools.partial(
    pl.kernel, mesh=_mesh,
    compiler_params=pltpu.CompilerParams(use_tc_tiling_on_sc=False),
    out_type=(jax.ShapeDtypeStruct((E,), jnp.float32),
              jax.ShapeDtypeStruct((SC_CORES, N1), jnp.float32)),
    scratch_types=[
        pltpu.VMEM_SHARED((N1,), jnp.float32),
        pltpu.VMEM((CHUNK,), jnp.int32),
        pltpu.VMEM((CHUNK,), jnp.int32),
        pltpu.VMEM((CHUNK,), jnp.float32),
        pltpu.VMEM((CHUNK,), jnp.float32),
        pltpu.VMEM((CHUNK,), jnp.float32),
        pltpu.VMEM((CHUNK,), jnp.float32),
        pltpu.VMEM((CHUNK,), jnp.int32),
        pltpu.VMEM((CHUNK,), jnp.int32),
        pltpu.VMEM((CHUNK,), jnp.float32),
        pltpu.VMEM((CHUNK,), jnp.float32),
        pltpu.VMEM((CHUNK,), jnp.float32),
        pltpu.VMEM((CHUNK,), jnp.float32),
        pltpu.SemaphoreType.DMA,
        pltpu.SemaphoreType.DMA,
        pltpu.SemaphoreType.DMA,
        pltpu.SemaphoreType.DMA,
        pltpu.SemaphoreType.DMA,
        pltpu.SemaphoreType.DMA,
        pltpu.SemaphoreType.DMA,
        pltpu.SemaphoreType.DMA,
    ],
)
def _sc_edge_prob(a_hbm, b_hbm, src_hbm, dst_hbm, ew_hbm, zeros_hbm,
                  w_hbm, deg_hbm,
                  deg_sp,
                  sv0, dv0, ev0, ag0, bg0, wv0,
                  sv1, dv1, ev1, ag1, bg1, wv1,
                  semI0, semI1, semG0, semG1, semW0, semW1, semD0, semD1):
    cid, sid, wid = _worker_id()
    base = wid * EPW
    bufs = ((sv0, dv0, ev0, ag0, bg0, wv0, semI0, semG0, semW0, semD0),
            (sv1, dv1, ev1, ag1, bg1, wv1, semI1, semG1, semW1, semD1))

    def issue_idx(cc, b):
        sv, dv, ev, _, _, _, semI, _, _, _ = bufs[b]
        off = base + cc * CHUNK
        pltpu.async_copy(src_hbm.at[pl.ds(off, CHUNK)], sv, semI)
        pltpu.async_copy(dst_hbm.at[pl.ds(off, CHUNK)], dv, semI)
        pltpu.async_copy(ew_hbm.at[pl.ds(off, CHUNK)], ev, semI)

    def wait_idx(cc, b):
        sv, dv, ev, _, _, _, semI, _, _, _ = bufs[b]
        off = base + cc * CHUNK
        pltpu.make_async_copy(src_hbm.at[pl.ds(off, CHUNK)], sv, semI).wait()
        pltpu.make_async_copy(dst_hbm.at[pl.ds(off, CHUNK)], dv, semI).wait()
        pltpu.make_async_copy(ew_hbm.at[pl.ds(off, CHUNK)], ev, semI).wait()

    def issue_gathers(b):
        sv, dv, _, ag, bg, _, _, semG, _, _ = bufs[b]
        pltpu.async_copy(a_hbm.at[sv], ag, semG)
        pltpu.async_copy(b_hbm.at[dv], bg, semG)

    def wait_gathers(b):
        sv, dv, _, ag, bg, _, _, semG, _, _ = bufs[b]
        pltpu.make_async_copy(a_hbm.at[sv], ag, semG).wait()
        pltpu.make_async_copy(b_hbm.at[dv], bg, semG).wait()

    def issue_w(cc, b):
        _, _, _, _, _, wv, _, _, semW, _ = bufs[b]
        off = base + cc * CHUNK
        pltpu.async_copy(wv, w_hbm.at[pl.ds(off, CHUNK)], semW)

    def wait_w(cc, b):
        _, _, _, _, _, wv, _, _, semW, _ = bufs[b]
        off = base + cc * CHUNK
        pltpu.make_async_copy(wv, w_hbm.at[pl.ds(off, CHUNK)], semW).wait()

    def issue_d(b):
        _, dv, _, _, _, wv, _, _, _, semD = bufs[b]
        pltpu.async_copy(wv, deg_sp.at[dv], semD, add=True)

    def wait_d(b):
        _, dv, _, _, _, wv, _, _, _, semD = bufs[b]
        pltpu.make_async_copy(wv, deg_sp.at[dv], semD).wait()

    def compute(b):
        _, _, ev, ag, bg, wv, _, _, _, _ = bufs[b]

        def vec_body(k, c2):
            s = pl.ds(k * 16, 16)
            t = ag[s] + bg[s]
            p = 1.0 / (1.0 + jnp.exp(-t))
            wv[s] = ev[s] * p
            return c2

        lax.fori_loop(0, CHUNK // 16, vec_body, 0)

    pltpu.sync_copy(zeros_hbm.at[pl.ds(sid * NPS1, NPS1)],
                    deg_sp.at[pl.ds(sid * NPS1, NPS1)])
    plsc.subcore_barrier()

    issue_idx(0, 0)
    wait_idx(0, 0)
    issue_gathers(0)

    def pair_body(j, carry):
        @pl.when(j >= 1)
        def _():
            wait_d(1)
            wait_w(2 * j - 1, 1)
        issue_idx(2 * j + 1, 1)
        wait_gathers(0)
        compute(0)
        wait_idx(2 * j + 1, 1)
        issue_gathers(1)
        issue_w(2 * j, 0)
        issue_d(0)
        wait_d(0)
        wait_w(2 * j, 0)
        issue_idx(2 * j + 2, 0)
        wait_gathers(1)
        compute(1)
        wait_idx(2 * j + 2, 0)
        issue_gathers(0)
        issue_w(2 * j + 1, 1)
        issue_d(1)
        return carry

    lax.fori_loop(0, (NCH - 1) // 2, pair_body, 0)
    wait_d(1)
    wait_w(NCH - 2, 1)
    wait_gathers(0)
    compute(0)
    issue_w(NCH - 1, 0)
    issue_d(0)
    wait_d(0)
    wait_w(NCH - 1, 0)
    plsc.subcore_barrier()
    pltpu.sync_copy(deg_sp.at[pl.ds(sid * NPS1, NPS1)],
                    deg_hbm.at[cid, pl.ds(sid * NPS1, NPS1)])


CHUNK_G = 720
NCH_G = EPW // CHUNK_G


@functools.partial(
    pl.kernel, mesh=_mesh,
    compiler_params=pltpu.CompilerParams(use_tc_tiling_on_sc=False),
    out_type=jax.ShapeDtypeStruct((SC_CORES, N1, HID), jnp.float32),
    scratch_types=[
        pltpu.VMEM_SHARED((N1, HID), jnp.float32),
        pltpu.VMEM((CHUNK_G,), jnp.int32),
        pltpu.VMEM((CHUNK_G,), jnp.int32),
        pltpu.VMEM((CHUNK_G,), jnp.float32),
        pltpu.VMEM((CHUNK_G, HID), jnp.float32),
        pltpu.VMEM((CHUNK_G,), jnp.int32),
        pltpu.VMEM((CHUNK_G,), jnp.int32),
        pltpu.VMEM((CHUNK_G,), jnp.float32),
        pltpu.VMEM((CHUNK_G, HID), jnp.float32),
        pltpu.SemaphoreType.DMA,
        pltpu.SemaphoreType.DMA,
        pltpu.SemaphoreType.DMA,
        pltpu.SemaphoreType.DMA,
        pltpu.SemaphoreType.DMA,
        pltpu.SemaphoreType.DMA,
    ],
)
def _sc_agg(h_hbm, src_hbm, dst_hbm, norm_hbm, zseg_hbm,
            part_hbm,
            acc_sp, sv0, dv0, nv0, rows0, sv1, dv1, nv1, rows1,
            semI0, semI1, semG0, semG1, semS0, semS1):
    cid, sid, wid = _worker_id()
    base = wid * EPW
    bufs = ((sv0, dv0, nv0, rows0, semI0, semG0, semS0),
            (sv1, dv1, nv1, rows1, semI1, semG1, semS1))

    def issue_idx(cc, b):
        sv, dv, nv, _, semI, _, _ = bufs[b]
        off = base + cc * CHUNK_G
        pltpu.async_copy(src_hbm.at[pl.ds(off, CHUNK_G)], sv, semI)
        pltpu.async_copy(dst_hbm.at[pl.ds(off, CHUNK_G)], dv, semI)
        pltpu.async_copy(norm_hbm.at[pl.ds(off, CHUNK_G)], nv, semI)

    def wait_idx(cc, b):
        sv, dv, nv, _, semI, _, _ = bufs[b]
        off = base + cc * CHUNK_G
        pltpu.make_async_copy(src_hbm.at[pl.ds(off, CHUNK_G)], sv, semI).wait()
        pltpu.make_async_copy(dst_hbm.at[pl.ds(off, CHUNK_G)], dv, semI).wait()
        pltpu.make_async_copy(norm_hbm.at[pl.ds(off, CHUNK_G)], nv, semI).wait()

    def issue_gather(b):
        sv, _, _, rows, _, semG, _ = bufs[b]
        pltpu.async_copy(h_hbm.at[sv], rows, semG)

    def wait_gather(b):
        sv, _, _, rows, _, semG, _ = bufs[b]
        pltpu.make_async_copy(h_hbm.at[sv], rows, semG).wait()

    def issue_scat(b):
        _, dv, _, rows, _, _, semS = bufs[b]
        pltpu.async_copy(rows, acc_sp.at[dv], semS, add=True)

    def wait_scat(b):
        _, dv, _, rows, _, _, semS = bufs[b]
        pltpu.make_async_copy(rows, acc_sp.at[dv], semS).wait()

    def scale(b):
        _, _, nv, rows, _, _, _ = bufs[b]

        def scale_body(k, c2):
            nvec = nv[pl.ds(k * 16, 16)]
            for i in range(16):
                e = k * 16 + i
                rows[e] = rows[e] * jnp.full((16,), nvec[i], jnp.float32)
            return c2

        lax.fori_loop(0, CHUNK_G // 16, scale_body, 0)

    pltpu.sync_copy(zseg_hbm.at[pl.ds(sid * NPS1, NPS1)],
                    acc_sp.at[pl.ds(sid * NPS1, NPS1)])
    plsc.subcore_barrier()

    issue_idx(0, 0)
    wait_idx(0, 0)
    issue_gather(0)

    def pair_body(j, carry):
        @pl.when(j >= 1)
        def _():
            wait_scat(1)
        issue_idx(2 * j + 1, 1)
        wait_gather(0)
        scale(0)
        wait_idx(2 * j + 1, 1)
        issue_gather(1)
        issue_scat(0)
        wait_scat(0)
        issue_idx(2 * j + 2, 0)
        wait_gather(1)
        scale(1)
        wait_idx(2 * j + 2, 0)
        issue_gather(0)
        issue_scat(1)
        return carry

    lax.fori_loop(0, (NCH_G - 1) // 2, pair_body, 0)
    wait_scat(1)
    wait_gather(0)
    scale(0)
    issue_scat(0)
    wait_scat(0)
    plsc.subcore_barrier()
    pltpu.sync_copy(acc_sp.at[pl.ds(sid * NPS1, NPS1)],
                    part_hbm.at[cid, pl.ds(sid * NPS1, NPS1)])


def kernel(x, edge_index, edge_weight, temperature, prob, prob_bias,
           W1, b1, W2, b2, W3, b3, lin1_W, lin1_b, lin2_W, lin2_b):
    src = edge_index[0]
    dst = edge_index[1]
    xf = (x.reshape(BATCH, ROIS, H0) * prob[None, :, :]).reshape(N, H0)
    pb = prob_bias[:, 0]
    a = xf @ pb[:H0]
    bvec = xf @ pb[H0:]
    zeros1 = jnp.zeros((N1,), jnp.float32)
    zseg = jnp.zeros((N1, HID), jnp.float32)

    w, deg2 = _sc_edge_prob(a, bvec, src, dst, edge_weight, zeros1)
    deg = deg2[0, :N] + deg2[1, :N] + 1.0
    dis = jnp.where(deg > 0, lax.rsqrt(jnp.maximum(deg, 1e-12)), 0.0)
    dis2 = dis * dis

    xs = []
    xk = xf
    for W, b in ((W1, b1), (W2, b2), (W3, b3)):
        h = xk @ W
        hd = dis[:, None] * h
        part = _sc_agg(hd, src, dst, w, zseg)
        acc = (part[0] + part[1])[:N]
        xk = jax.nn.relu(dis[:, None] * (acc + hd) + b)
        xs.append(xk)

    xcat = jnp.concatenate(xs, axis=-1)
    feat = xcat.reshape(BATCH, ROIS * NLAYERS * HID)
    hdense = jax.nn.relu(feat @ lin1_W + lin1_b)
    return hdense @ lin2_W + lin2_b
